# Initial kernel scaffold; baseline (speedup 1.0000x reference)
import jax
import jax.numpy as jnp
from jax.experimental import pallas as pl


def kernel(x, centroids):
    raise NotImplementedError("write your pallas kernel here")


# speedup vs baseline: 1.1769x; 1.1769x over previous
import jax
import jax.numpy as jnp
from jax.experimental import pallas as pl
from jax.experimental.pallas import tpu as pltpu

B = 9216
D = 256
K = 8192
BT = 1024
KT = 1024
NB = B // BT
NK = K // KT


def _vq_body(x_ref, ct_ref, x2_ref, c2_ref, o_ref, mv_ref, mi_ref):
    j = pl.program_id(1)

    @pl.when(j == 0)
    def _init():
        mv_ref[...] = jnp.full((BT, 1), jnp.inf, jnp.float32)
        mi_ref[...] = jnp.zeros((BT, 1), jnp.int32)

    s = jax.lax.dot_general(
        x_ref[...].astype(jnp.bfloat16), ct_ref[...].astype(jnp.bfloat16),
        (((1,), (0,)), ((), ())),
        preferred_element_type=jnp.float32,
    )
    t = (x2_ref[...] - 2.0 * s) + c2_ref[...]

    tmin = jnp.min(t, axis=1, keepdims=True)
    idx = jax.lax.broadcasted_iota(jnp.int32, (BT, KT), 1) + j * KT
    tidx = jnp.min(jnp.where(t == tmin, idx, K), axis=1, keepdims=True)

    v = jnp.sqrt(jnp.maximum(tmin, 0.0))
    better = v < mv_ref[...]
    mi_ref[...] = jnp.where(better, tidx, mi_ref[...])
    bits = jax.lax.bitcast_convert_type(v, jnp.uint32)
    bits = (bits + 0x7FFF + ((bits >> 16) & 1)) & jnp.uint32(0xFFFF0000)
    vb = jax.lax.bitcast_convert_type(bits, jnp.float32)
    mv_ref[...] = jnp.where(better, vb, mv_ref[...])

    @pl.when(j == NK - 1)
    def _emit():
        o_ref[...] = mi_ref[...]


def kernel(x, centroids):
    ct = centroids.T
    x2 = jnp.sum(x * x, axis=1)[:, None]
    c2 = jnp.sum(centroids * centroids, axis=1)[None, :]
    out = pl.pallas_call(
        _vq_body,
        grid=(NB, NK),
        in_specs=[
            pl.BlockSpec((BT, D), lambda i, j: (i, 0)),
            pl.BlockSpec((D, KT), lambda i, j: (0, j)),
            pl.BlockSpec((BT, 1), lambda i, j: (i, 0)),
            pl.BlockSpec((1, KT), lambda i, j: (0, j)),
        ],
        out_specs=pl.BlockSpec((BT, 1), lambda i, j: (i, 0)),
        out_shape=jax.ShapeDtypeStruct((B, 1), jnp.int32),
        scratch_shapes=[
            pltpu.VMEM((BT, 1), jnp.float32),
            pltpu.VMEM((BT, 1), jnp.int32),
        ],
        compiler_params=pltpu.CompilerParams(
            dimension_semantics=("parallel", "arbitrary"),
        ),
    )(x, ct, x2, c2)
    return out.reshape(B)
